# initial kernel scaffold (unmeasured)
import jax
import jax.numpy as jnp
from jax import lax
from jax.experimental import pallas as pl
from jax.experimental.pallas import tpu as pltpu

N_DEV = 4


def kernel(x, w_mat):
    m_per, k = x.shape
    _, n_per = w_mat.shape

    def body(x_ref, w_ref, out_ref, xg_ref, wb_ref, sc_ref,
             send_sems, recv_sems, s_send_sems, s_recv_sems):
        my = lax.axis_index("i")
        left = lax.rem(my + N_DEV - 1, N_DEV)
        right = lax.rem(my + 1, N_DEV)

        barrier_sem = pltpu.get_barrier_semaphore()
        for nbr in (left, right):
            pl.semaphore_signal(
                barrier_sem, inc=1,
                device_id=(nbr,), device_id_type=pl.DeviceIdType.MESH,
            )
        pl.semaphore_wait(barrier_sem, 2)

        wb_ref[...] = w_ref[...].astype(jnp.bfloat16)
        xg_ref[0] = x_ref[...].astype(jnp.bfloat16)

        def gemm(slot, origin):
            out_ref[pl.ds(origin * m_per, m_per), :] = jax.lax.dot_general(
                xg_ref[slot], wb_ref[...],
                (((1,), (0,)), ((), ())),
                preferred_element_type=jnp.float32,
            )

        gemm(0, my)

        for h in range(N_DEV - 1):
            rdma = pltpu.make_async_remote_copy(
                src_ref=xg_ref.at[h],
                dst_ref=xg_ref.at[h + 1],
                send_sem=send_sems.at[h],
                recv_sem=recv_sems.at[h],
                device_id=(right,),
                device_id_type=pl.DeviceIdType.MESH,
            )
            rdma.start()
            rdma.wait()
            gemm(h + 1, lax.rem(my + N_DEV - 1 - h, N_DEV))

        amax = jnp.max(jnp.abs(out_ref[...]))
        sc_ref[0] = jnp.full((8, 128), amax, jnp.float32)
        for h in range(N_DEV - 1):
            rdma = pltpu.make_async_remote_copy(
                src_ref=sc_ref.at[h],
                dst_ref=sc_ref.at[h + 1],
                send_sem=s_send_sems.at[h],
                recv_sem=s_recv_sems.at[h],
                device_id=(right,),
                device_id_type=pl.DeviceIdType.MESH,
            )
            rdma.start()
            rdma.wait()
        gmax = jnp.max(sc_ref[...])

        scale = gmax / 448.0
        y = out_ref[...]
        q = jnp.clip(y * (448.0 / gmax), -448.0, 448.0)
        out_ref[...] = q.astype(jnp.float8_e4m3fn).astype(jnp.float32) * scale

    return pl.pallas_call(
        body,
        out_shape=jax.ShapeDtypeStruct((N_DEV * m_per, n_per), jnp.float32),
        in_specs=[
            pl.BlockSpec(memory_space=pltpu.VMEM),
            pl.BlockSpec(memory_space=pltpu.VMEM),
        ],
        out_specs=pl.BlockSpec(memory_space=pltpu.VMEM),
        scratch_shapes=[
            pltpu.VMEM((N_DEV, m_per, k), jnp.bfloat16),
            pltpu.VMEM((k, n_per), jnp.bfloat16),
            pltpu.VMEM((N_DEV, 8, 128), jnp.float32),
            pltpu.SemaphoreType.DMA((N_DEV - 1,)),
            pltpu.SemaphoreType.DMA((N_DEV - 1,)),
            pltpu.SemaphoreType.DMA((N_DEV - 1,)),
            pltpu.SemaphoreType.DMA((N_DEV - 1,)),
        ],
        compiler_params=pltpu.CompilerParams(collective_id=0),
    )(x, w_mat)


# baseline (device time: 334477 ns/iter reference)
import jax
import jax.numpy as jnp
from jax import lax
from jax.experimental import pallas as pl
from jax.experimental.pallas import tpu as pltpu

N_DEV = 4


def kernel(x, w_mat):
    m_per, k = x.shape
    _, n_per = w_mat.shape
    xb = x.astype(jnp.bfloat16)
    wb = w_mat.astype(jnp.bfloat16)

    def body(x_ref, w_ref, out_ref, xg_ref, sc_ref,
             send_sems, recv_sems, s_send_sems, s_recv_sems):
        my = lax.axis_index("i")
        left = lax.rem(my + N_DEV - 1, N_DEV)
        right = lax.rem(my + 1, N_DEV)

        barrier_sem = pltpu.get_barrier_semaphore()
        for nbr in (left, right):
            pl.semaphore_signal(
                barrier_sem, inc=1,
                device_id=(nbr,), device_id_type=pl.DeviceIdType.MESH,
            )
        pl.semaphore_wait(barrier_sem, 2)

        def gemm(src, origin):
            out_ref[pl.ds(origin * m_per, m_per), :] = jax.lax.dot_general(
                src, w_ref[...],
                (((1,), (0,)), ((), ())),
                preferred_element_type=jnp.float32,
            )

        gemm(x_ref[...], my)

        for h in range(N_DEV - 1):
            rdma = pltpu.make_async_remote_copy(
                src_ref=x_ref if h == 0 else xg_ref.at[h - 1],
                dst_ref=xg_ref.at[h],
                send_sem=send_sems.at[h],
                recv_sem=recv_sems.at[h],
                device_id=(right,),
                device_id_type=pl.DeviceIdType.MESH,
            )
            rdma.start()
            rdma.wait()
            gemm(xg_ref[h], lax.rem(my + N_DEV - 1 - h, N_DEV))

        amax = jnp.float32(0.0)
        for i in range(N_DEV):
            amax = jnp.maximum(
                amax, jnp.max(jnp.abs(out_ref[pl.ds(i * m_per, m_per), :]))
            )
        sc_ref[0] = jnp.full((8, 128), amax, jnp.float32)
        for h in range(N_DEV - 1):
            rdma = pltpu.make_async_remote_copy(
                src_ref=sc_ref.at[h],
                dst_ref=sc_ref.at[h + 1],
                send_sem=s_send_sems.at[h],
                recv_sem=s_recv_sems.at[h],
                device_id=(right,),
                device_id_type=pl.DeviceIdType.MESH,
            )
            rdma.start()
            rdma.wait()
        gmax = jnp.max(sc_ref[...])

        scale = gmax / 448.0
        inv = 448.0 / gmax
        for i in range(N_DEV):
            rows = pl.ds(i * m_per, m_per)
            q = jnp.clip(out_ref[rows, :] * inv, -448.0, 448.0)
            out_ref[rows, :] = (
                q.astype(jnp.float8_e4m3fn).astype(jnp.float32) * scale
            )

    return pl.pallas_call(
        body,
        out_shape=jax.ShapeDtypeStruct((N_DEV * m_per, n_per), jnp.float32),
        in_specs=[
            pl.BlockSpec(memory_space=pltpu.VMEM),
            pl.BlockSpec(memory_space=pltpu.VMEM),
        ],
        out_specs=pl.BlockSpec(memory_space=pltpu.VMEM),
        scratch_shapes=[
            pltpu.VMEM((N_DEV - 1, m_per, k), jnp.bfloat16),
            pltpu.VMEM((N_DEV, 8, 128), jnp.float32),
            pltpu.SemaphoreType.DMA((N_DEV - 1,)),
            pltpu.SemaphoreType.DMA((N_DEV - 1,)),
            pltpu.SemaphoreType.DMA((N_DEV - 1,)),
            pltpu.SemaphoreType.DMA((N_DEV - 1,)),
        ],
        compiler_params=pltpu.CompilerParams(
            collective_id=0,
            vmem_limit_bytes=63 * 1024 * 1024,
        ),
    )(xb, wb)


# device time: 185440 ns/iter; 1.8037x vs baseline; 1.8037x over previous
import jax
import jax.numpy as jnp
from jax import lax
from jax.experimental import pallas as pl
from jax.experimental.pallas import tpu as pltpu

N_DEV = 4


def kernel(x, w_mat):
    m_per, k = x.shape
    _, n_per = w_mat.shape
    m_half = m_per // 2
    xb = x.astype(jnp.bfloat16)
    wb = w_mat.astype(jnp.bfloat16)

    def body(x_ref, w_ref, out_ref, bufL, bufR, bufD, sc_ref,
             send_sems, recv_sems, s_send_sems, s_recv_sems):
        my = lax.axis_index("i")
        left = lax.rem(my + N_DEV - 1, N_DEV)
        right = lax.rem(my + 1, N_DEV)

        barrier_sem = pltpu.get_barrier_semaphore()
        for nbr in (left, right):
            pl.semaphore_signal(
                barrier_sem, inc=1,
                device_id=(nbr,), device_id_type=pl.DeviceIdType.MESH,
            )
        pl.semaphore_wait(barrier_sem, 2)

        send_r1 = pltpu.make_async_remote_copy(
            src_ref=x_ref, dst_ref=bufL,
            send_sem=send_sems.at[0], recv_sem=recv_sems.at[0],
            device_id=(right,), device_id_type=pl.DeviceIdType.MESH,
        )
        send_l1 = pltpu.make_async_remote_copy(
            src_ref=x_ref, dst_ref=bufR,
            send_sem=send_sems.at[1], recv_sem=recv_sems.at[1],
            device_id=(left,), device_id_type=pl.DeviceIdType.MESH,
        )
        send_r1.start()
        send_l1.start()

        def gemm(src, origin):
            out_ref[pl.ds(origin * m_per, m_per), :] = jax.lax.dot_general(
                src, w_ref[...],
                (((1,), (0,)), ((), ())),
                preferred_element_type=jnp.float32,
            )

        def chunk_amax(origin):
            return jnp.max(jnp.abs(out_ref[pl.ds(origin * m_per, m_per), :]))

        gemm(x_ref[...], my)
        amax = chunk_amax(my)

        send_r1.wait_recv()
        send_r2 = pltpu.make_async_remote_copy(
            src_ref=bufL.at[pl.ds(0, m_half), :],
            dst_ref=bufD.at[pl.ds(0, m_half), :],
            send_sem=send_sems.at[2], recv_sem=recv_sems.at[2],
            device_id=(right,), device_id_type=pl.DeviceIdType.MESH,
        )
        send_r2.start()
        origin_l = lax.rem(my + N_DEV - 1, N_DEV)
        gemm(bufL[...], origin_l)
        amax = jnp.maximum(amax, chunk_amax(origin_l))

        send_l1.wait_recv()
        send_l2 = pltpu.make_async_remote_copy(
            src_ref=bufR.at[pl.ds(m_half, m_half), :],
            dst_ref=bufD.at[pl.ds(m_half, m_half), :],
            send_sem=send_sems.at[3], recv_sem=recv_sems.at[3],
            device_id=(left,), device_id_type=pl.DeviceIdType.MESH,
        )
        send_l2.start()
        origin_r = lax.rem(my + 1, N_DEV)
        gemm(bufR[...], origin_r)
        amax = jnp.maximum(amax, chunk_amax(origin_r))

        send_r2.wait_recv()
        send_l2.wait_recv()
        origin_d = lax.rem(my + 2, N_DEV)
        gemm(bufD[...], origin_d)
        amax = jnp.maximum(amax, chunk_amax(origin_d))

        send_r1.wait_send()
        send_l1.wait_send()
        send_r2.wait_send()
        send_l2.wait_send()

        sc_ref[0] = jnp.full((8, 128), amax, jnp.float32)
        s_r1 = pltpu.make_async_remote_copy(
            src_ref=sc_ref.at[0], dst_ref=sc_ref.at[1],
            send_sem=s_send_sems.at[0], recv_sem=s_recv_sems.at[0],
            device_id=(right,), device_id_type=pl.DeviceIdType.MESH,
        )
        s_l1 = pltpu.make_async_remote_copy(
            src_ref=sc_ref.at[0], dst_ref=sc_ref.at[2],
            send_sem=s_send_sems.at[1], recv_sem=s_recv_sems.at[1],
            device_id=(left,), device_id_type=pl.DeviceIdType.MESH,
        )
        s_r1.start()
        s_l1.start()
        s_r1.wait_recv()
        s_r2 = pltpu.make_async_remote_copy(
            src_ref=sc_ref.at[1], dst_ref=sc_ref.at[3],
            send_sem=s_send_sems.at[2], recv_sem=s_recv_sems.at[2],
            device_id=(right,), device_id_type=pl.DeviceIdType.MESH,
        )
        s_r2.start()
        s_l1.wait_recv()
        s_l2 = pltpu.make_async_remote_copy(
            src_ref=sc_ref.at[2], dst_ref=sc_ref.at[4],
            send_sem=s_send_sems.at[3], recv_sem=s_recv_sems.at[3],
            device_id=(left,), device_id_type=pl.DeviceIdType.MESH,
        )
        s_l2.start()
        s_r2.wait_recv()
        s_l2.wait_recv()
        s_r1.wait_send()
        s_l1.wait_send()
        s_r2.wait_send()
        s_l2.wait_send()
        gmax = jnp.max(sc_ref[...])

        scale = gmax / 448.0
        inv = 448.0 / gmax
        for i in range(N_DEV):
            rows = pl.ds(i * m_per, m_per)
            q = jnp.clip(out_ref[rows, :] * inv, -448.0, 448.0)
            out_ref[rows, :] = (
                q.astype(jnp.float8_e4m3fn).astype(jnp.float32) * scale
            )

    return pl.pallas_call(
        body,
        out_shape=jax.ShapeDtypeStruct((N_DEV * m_per, n_per), jnp.float32),
        in_specs=[
            pl.BlockSpec(memory_space=pltpu.VMEM),
            pl.BlockSpec(memory_space=pltpu.VMEM),
        ],
        out_specs=pl.BlockSpec(memory_space=pltpu.VMEM),
        scratch_shapes=[
            pltpu.VMEM((m_per, k), jnp.bfloat16),
            pltpu.VMEM((m_per, k), jnp.bfloat16),
            pltpu.VMEM((m_per, k), jnp.bfloat16),
            pltpu.VMEM((5, 8, 128), jnp.float32),
            pltpu.SemaphoreType.DMA((4,)),
            pltpu.SemaphoreType.DMA((4,)),
            pltpu.SemaphoreType.DMA((4,)),
            pltpu.SemaphoreType.DMA((4,)),
        ],
        compiler_params=pltpu.CompilerParams(
            collective_id=0,
            vmem_limit_bytes=63 * 1024 * 1024,
        ),
    )(xb, wb)


# device time: 177794 ns/iter; 1.8813x vs baseline; 1.0430x over previous
import jax
import jax.numpy as jnp
from jax import lax
from jax.experimental import pallas as pl
from jax.experimental.pallas import tpu as pltpu

N_DEV = 4
STAGE_ROWS = 256


def kernel(x, w_mat):
    m_per, k = x.shape
    _, n_per = w_mat.shape
    m_half = m_per // 2

    def body(x_hbm, w_ref, out_ref, xb, wb, stg, bufL, bufR, bufD, sc_ref,
             stg_sem, send_sems, recv_sems, s_send_sems, s_recv_sems):
        my = lax.axis_index("i")
        left = lax.rem(my + N_DEV - 1, N_DEV)
        right = lax.rem(my + 1, N_DEV)
        diag = lax.rem(my + 2, N_DEV)

        n_stage = m_per // STAGE_ROWS
        for i in range(n_stage):
            rows = pl.ds(i * STAGE_ROWS, STAGE_ROWS)
            cp = pltpu.make_async_copy(x_hbm.at[rows, :], stg, stg_sem)
            cp.start()
            cp.wait()
            xb[rows, :] = stg[...].astype(jnp.bfloat16)

        barrier_sem = pltpu.get_barrier_semaphore()
        for nbr in (left, right):
            pl.semaphore_signal(
                barrier_sem, inc=1,
                device_id=(nbr,), device_id_type=pl.DeviceIdType.MESH,
            )
        pl.semaphore_wait(barrier_sem, 2)

        send_r1 = pltpu.make_async_remote_copy(
            src_ref=xb, dst_ref=bufL,
            send_sem=send_sems.at[0], recv_sem=recv_sems.at[0],
            device_id=(right,), device_id_type=pl.DeviceIdType.MESH,
        )
        send_l1 = pltpu.make_async_remote_copy(
            src_ref=xb, dst_ref=bufR,
            send_sem=send_sems.at[1], recv_sem=recv_sems.at[1],
            device_id=(left,), device_id_type=pl.DeviceIdType.MESH,
        )
        send_r1.start()
        send_l1.start()

        wb[...] = w_ref[...].astype(jnp.bfloat16)

        def gemm(src, origin):
            out_ref[pl.ds(origin * m_per, m_per), :] = jax.lax.dot_general(
                src, wb[...],
                (((1,), (0,)), ((), ())),
                preferred_element_type=jnp.float32,
            )

        def chunk_amax(origin):
            return jnp.max(jnp.abs(out_ref[pl.ds(origin * m_per, m_per), :]))

        gemm(xb[...], my)
        amax = chunk_amax(my)

        send_r1.wait_recv()
        send_r2 = pltpu.make_async_remote_copy(
            src_ref=bufL.at[pl.ds(0, m_half), :],
            dst_ref=bufD.at[pl.ds(0, m_half), :],
            send_sem=send_sems.at[2], recv_sem=recv_sems.at[2],
            device_id=(right,), device_id_type=pl.DeviceIdType.MESH,
        )
        send_r2.start()
        gemm(bufL[...], left)
        amax = jnp.maximum(amax, chunk_amax(left))

        send_l1.wait_recv()
        send_l2 = pltpu.make_async_remote_copy(
            src_ref=bufR.at[pl.ds(m_half, m_half), :],
            dst_ref=bufD.at[pl.ds(m_half, m_half), :],
            send_sem=send_sems.at[3], recv_sem=recv_sems.at[3],
            device_id=(left,), device_id_type=pl.DeviceIdType.MESH,
        )
        send_l2.start()
        gemm(bufR[...], right)
        amax = jnp.maximum(amax, chunk_amax(right))

        send_r2.wait_recv()
        send_l2.wait_recv()
        gemm(bufD[...], diag)
        amax = jnp.maximum(amax, chunk_amax(diag))

        send_r1.wait_send()
        send_l1.wait_send()
        send_r2.wait_send()
        send_l2.wait_send()

        sc_ref[0] = jnp.full((8, 128), amax, jnp.float32)
        s_r = pltpu.make_async_remote_copy(
            src_ref=sc_ref.at[0], dst_ref=sc_ref.at[1],
            send_sem=s_send_sems.at[0], recv_sem=s_recv_sems.at[0],
            device_id=(right,), device_id_type=pl.DeviceIdType.MESH,
        )
        s_l = pltpu.make_async_remote_copy(
            src_ref=sc_ref.at[0], dst_ref=sc_ref.at[2],
            send_sem=s_send_sems.at[1], recv_sem=s_recv_sems.at[1],
            device_id=(left,), device_id_type=pl.DeviceIdType.MESH,
        )
        s_d = pltpu.make_async_remote_copy(
            src_ref=sc_ref.at[0], dst_ref=sc_ref.at[3],
            send_sem=s_send_sems.at[2], recv_sem=s_recv_sems.at[2],
            device_id=(diag,), device_id_type=pl.DeviceIdType.MESH,
        )
        s_r.start()
        s_l.start()
        s_d.start()
        s_r.wait_recv()
        s_l.wait_recv()
        s_d.wait_recv()
        s_r.wait_send()
        s_l.wait_send()
        s_d.wait_send()
        gmax = jnp.max(sc_ref[...])

        scale = gmax / 448.0
        inv = 448.0 / gmax
        for i in range(N_DEV):
            rows = pl.ds(i * m_per, m_per)
            q = jnp.clip(out_ref[rows, :] * inv, -448.0, 448.0)
            out_ref[rows, :] = (
                q.astype(jnp.float8_e4m3fn).astype(jnp.float32) * scale
            )

    return pl.pallas_call(
        body,
        out_shape=jax.ShapeDtypeStruct((N_DEV * m_per, n_per), jnp.float32),
        in_specs=[
            pl.BlockSpec(memory_space=pl.ANY),
            pl.BlockSpec(memory_space=pltpu.VMEM),
        ],
        out_specs=pl.BlockSpec(memory_space=pltpu.VMEM),
        scratch_shapes=[
            pltpu.VMEM((m_per, k), jnp.bfloat16),
            pltpu.VMEM((k, n_per), jnp.bfloat16),
            pltpu.VMEM((STAGE_ROWS, k), jnp.float32),
            pltpu.VMEM((m_per, k), jnp.bfloat16),
            pltpu.VMEM((m_per, k), jnp.bfloat16),
            pltpu.VMEM((m_per, k), jnp.bfloat16),
            pltpu.VMEM((4, 8, 128), jnp.float32),
            pltpu.SemaphoreType.DMA,
            pltpu.SemaphoreType.DMA((4,)),
            pltpu.SemaphoreType.DMA((4,)),
            pltpu.SemaphoreType.DMA((3,)),
            pltpu.SemaphoreType.DMA((3,)),
        ],
        compiler_params=pltpu.CompilerParams(
            collective_id=0,
            vmem_limit_bytes=63 * 1024 * 1024,
        ),
    )(x, w_mat)


# device time: 174422 ns/iter; 1.9176x vs baseline; 1.0193x over previous
import jax
import jax.numpy as jnp
from jax import lax
from jax.experimental import pallas as pl
from jax.experimental.pallas import tpu as pltpu

N_DEV = 4
STAGE_ROWS = 256


def kernel(x, w_mat):
    m_per, k = x.shape
    _, n_per = w_mat.shape
    m_half = m_per // 2

    n_stage = m_per // STAGE_ROWS

    def body(x_hbm, w_ref, out_ref, xb, wb, stg, bufL, bufR, bufD, sc_ref,
             stg_sems, send_sems, recv_sems, s_send_sems, s_recv_sems):
        my = lax.axis_index("i")
        left = lax.rem(my + N_DEV - 1, N_DEV)
        right = lax.rem(my + 1, N_DEV)
        diag = lax.rem(my + 2, N_DEV)

        barrier_sem = pltpu.get_barrier_semaphore()
        for nbr in (left, right):
            pl.semaphore_signal(
                barrier_sem, inc=1,
                device_id=(nbr,), device_id_type=pl.DeviceIdType.MESH,
            )
        pl.semaphore_wait(barrier_sem, 2)

        def stage_dma(i):
            cp = pltpu.make_async_copy(
                x_hbm.at[pl.ds(i * STAGE_ROWS, STAGE_ROWS), :],
                stg.at[i % 2], stg_sems.at[i % 2],
            )
            cp.start()
            return cp

        dma = stage_dma(0)
        for i in range(n_stage):
            nxt = stage_dma(i + 1) if i + 1 < n_stage else None
            dma.wait()
            rows = pl.ds(i * STAGE_ROWS, STAGE_ROWS)
            xb[rows, :] = stg[i % 2].astype(jnp.bfloat16)
            dma = nxt

        send_r1 = pltpu.make_async_remote_copy(
            src_ref=xb, dst_ref=bufL,
            send_sem=send_sems.at[0], recv_sem=recv_sems.at[0],
            device_id=(right,), device_id_type=pl.DeviceIdType.MESH,
        )
        send_l1 = pltpu.make_async_remote_copy(
            src_ref=xb, dst_ref=bufR,
            send_sem=send_sems.at[1], recv_sem=recv_sems.at[1],
            device_id=(left,), device_id_type=pl.DeviceIdType.MESH,
        )
        send_r1.start()
        send_l1.start()

        wb[...] = w_ref[...].astype(jnp.bfloat16)

        def gemm(src, origin):
            out_ref[pl.ds(origin * m_per, m_per), :] = jax.lax.dot_general(
                src, wb[...],
                (((1,), (0,)), ((), ())),
                preferred_element_type=jnp.float32,
            )

        def chunk_amax(origin):
            return jnp.max(jnp.abs(out_ref[pl.ds(origin * m_per, m_per), :]))

        gemm(xb[...], my)
        amax = chunk_amax(my)

        send_r1.wait_recv()
        send_r2 = pltpu.make_async_remote_copy(
            src_ref=bufL.at[pl.ds(0, m_half), :],
            dst_ref=bufD.at[pl.ds(0, m_half), :],
            send_sem=send_sems.at[2], recv_sem=recv_sems.at[2],
            device_id=(right,), device_id_type=pl.DeviceIdType.MESH,
        )
        send_r2.start()
        gemm(bufL[...], left)
        amax = jnp.maximum(amax, chunk_amax(left))

        send_l1.wait_recv()
        send_l2 = pltpu.make_async_remote_copy(
            src_ref=bufR.at[pl.ds(m_half, m_half), :],
            dst_ref=bufD.at[pl.ds(m_half, m_half), :],
            send_sem=send_sems.at[3], recv_sem=recv_sems.at[3],
            device_id=(left,), device_id_type=pl.DeviceIdType.MESH,
        )
        send_l2.start()
        gemm(bufR[...], right)
        amax = jnp.maximum(amax, chunk_amax(right))

        send_r2.wait_recv()
        send_l2.wait_recv()
        gemm(bufD[...], diag)
        amax = jnp.maximum(amax, chunk_amax(diag))

        send_r1.wait_send()
        send_l1.wait_send()
        send_r2.wait_send()
        send_l2.wait_send()

        sc_ref[0] = jnp.full((8, 128), amax, jnp.float32)
        s_r = pltpu.make_async_remote_copy(
            src_ref=sc_ref.at[0], dst_ref=sc_ref.at[1],
            send_sem=s_send_sems.at[0], recv_sem=s_recv_sems.at[0],
            device_id=(right,), device_id_type=pl.DeviceIdType.MESH,
        )
        s_l = pltpu.make_async_remote_copy(
            src_ref=sc_ref.at[0], dst_ref=sc_ref.at[2],
            send_sem=s_send_sems.at[1], recv_sem=s_recv_sems.at[1],
            device_id=(left,), device_id_type=pl.DeviceIdType.MESH,
        )
        s_d = pltpu.make_async_remote_copy(
            src_ref=sc_ref.at[0], dst_ref=sc_ref.at[3],
            send_sem=s_send_sems.at[2], recv_sem=s_recv_sems.at[2],
            device_id=(diag,), device_id_type=pl.DeviceIdType.MESH,
        )
        s_r.start()
        s_l.start()
        s_d.start()
        s_r.wait_recv()
        s_l.wait_recv()
        s_d.wait_recv()
        s_r.wait_send()
        s_l.wait_send()
        s_d.wait_send()
        gmax = jnp.max(sc_ref[...])

        scale = gmax / 448.0
        inv = 448.0 / gmax
        for i in range(N_DEV):
            rows = pl.ds(i * m_per, m_per)
            q = jnp.clip(out_ref[rows, :] * inv, -448.0, 448.0)
            out_ref[rows, :] = (
                q.astype(jnp.float8_e4m3fn).astype(jnp.float32) * scale
            )

    return pl.pallas_call(
        body,
        out_shape=jax.ShapeDtypeStruct((N_DEV * m_per, n_per), jnp.float32),
        in_specs=[
            pl.BlockSpec(memory_space=pl.ANY),
            pl.BlockSpec(memory_space=pltpu.VMEM),
        ],
        out_specs=pl.BlockSpec(memory_space=pltpu.VMEM),
        scratch_shapes=[
            pltpu.VMEM((m_per, k), jnp.bfloat16),
            pltpu.VMEM((k, n_per), jnp.bfloat16),
            pltpu.VMEM((2, STAGE_ROWS, k), jnp.float32),
            pltpu.VMEM((m_per, k), jnp.bfloat16),
            pltpu.VMEM((m_per, k), jnp.bfloat16),
            pltpu.VMEM((m_per, k), jnp.bfloat16),
            pltpu.VMEM((4, 8, 128), jnp.float32),
            pltpu.SemaphoreType.DMA((2,)),
            pltpu.SemaphoreType.DMA((4,)),
            pltpu.SemaphoreType.DMA((4,)),
            pltpu.SemaphoreType.DMA((3,)),
            pltpu.SemaphoreType.DMA((3,)),
        ],
        compiler_params=pltpu.CompilerParams(
            collective_id=0,
            vmem_limit_bytes=63 * 1024 * 1024,
        ),
    )(x, w_mat)
